# Initial kernel scaffold; baseline (speedup 1.0000x reference)
import jax
import jax.numpy as jnp
from jax.experimental import pallas as pl


def kernel(user_input, num_idx, item_input, labels, embedding_Q, embedding_Q_, bias, W, b, h):
    raise NotImplementedError("write your pallas kernel here")


# speedup vs baseline: 8.2773x; 8.2773x over previous
import functools

import jax
import jax.numpy as jnp
from jax import lax
from jax.experimental import pallas as pl
from jax.experimental.pallas import tpu as pltpu
from jax.experimental.pallas import tpu_sc as plsc

_B = 4096
_HN = 50
_D = 64
_NI = 100000
_ALPHA = 0.5
_BETA = 0.5
_LAM = 1e-7
_GAM = 1e-7
_ETA = 1e-5

_UBLK = 32
_ASTEPS = 256 // _UBLK
_BBLK = _B // _ASTEPS

_NW = 32
_TOT = _B * _HN
_CHUNK = _TOT // _NW


def _tables_body(ef_ref, eb_ref, w_ref, b_ref, h_ref, ui_ref, ii_ref,
                 et_ref, eg_ref, flat_ref, sw_ref):
    i = pl.program_id(0)
    ef = ef_ref[...]
    eb = eb_ref[...]
    w = w_ref[...]
    p_all = lax.dot_general(ef, w, (((1,), (0,)), ((), ())),
                            precision=lax.Precision.HIGHEST,
                            preferred_element_type=jnp.float32)
    p_u = lax.dot_general(eb, w, (((1,), (0,)), ((), ())),
                          precision=lax.Precision.HIGHEST,
                          preferred_element_type=jnp.float32)
    pre = p_u[:, None, :] + p_all[None, :, :] + b_ref[...][None, :, :]
    sig = 1.0 / (1.0 + jnp.exp(-pre))
    t = jnp.sum(sig * h_ref[...][None, :, :], axis=2)
    et = jnp.exp(t)
    g = lax.dot_general(eb, ef, (((1,), (1,)), ((), ())),
                        precision=lax.Precision.HIGHEST,
                        preferred_element_type=jnp.float32)
    et_ref[...] = et
    eg_ref[...] = et * g
    flat_ref[...] = ui_ref[...] * 256 + ii_ref[...]

    @pl.when(i == 0)
    def _():
        sw_ref[0, 0] = jnp.sum(w * w)


_tables_call = pl.pallas_call(
    _tables_body,
    grid=(_ASTEPS,),
    in_specs=[
        pl.BlockSpec((256, _D), lambda i: (0, 0)),
        pl.BlockSpec((_UBLK, _D), lambda i: (i, 0)),
        pl.BlockSpec((_D, _D), lambda i: (0, 0)),
        pl.BlockSpec((1, _D), lambda i: (0, 0)),
        pl.BlockSpec((1, _D), lambda i: (0, 0)),
        pl.BlockSpec((_BBLK, _HN), lambda i: (i, 0)),
        pl.BlockSpec((_BBLK, 1), lambda i: (i, 0)),
    ],
    out_specs=[
        pl.BlockSpec((_UBLK, 256), lambda i: (i, 0)),
        pl.BlockSpec((_UBLK, 256), lambda i: (i, 0)),
        pl.BlockSpec((_BBLK, _HN), lambda i: (i, 0)),
        pl.BlockSpec(memory_space=pltpu.SMEM, block_shape=(1, 1),
                     index_map=lambda i: (0, 0)),
    ],
    out_shape=[
        jax.ShapeDtypeStruct((256, 256), jnp.float32),
        jax.ShapeDtypeStruct((256, 256), jnp.float32),
        jax.ShapeDtypeStruct((_B, _HN), jnp.int32),
        jax.ShapeDtypeStruct((1, 1), jnp.float32),
    ],
)


_SCHUNK = _TOT // 16


@functools.cache
def _make_sc_gather():
    mesh = plsc.VectorSubcoreMesh(core_axis_name="c", subcore_axis_name="s",
                                  num_cores=2, num_subcores=16)

    @functools.partial(
        pl.kernel,
        out_type=jax.ShapeDtypeStruct((2, _TOT), jnp.float32),
        mesh=mesh,
        scratch_types=[
            pltpu.VMEM((65536,), jnp.float32),
            pltpu.VMEM((_SCHUNK,), jnp.int32),
            pltpu.VMEM((_SCHUNK,), jnp.float32),
        ],
        compiler_params=pltpu.CompilerParams(use_tc_tiling_on_sc=False,
                                             needs_layout_passes=False),
    )
    def sc_gather(et_hbm, eg_hbm, flat_hbm, out_hbm, tab_v, idx_v, val_v):
        c = lax.axis_index("c")
        s = lax.axis_index("s")
        base = s * _SCHUNK

        @pl.when(c == 0)
        def _():
            pltpu.sync_copy(et_hbm, tab_v)

        @pl.when(c == 1)
        def _():
            pltpu.sync_copy(eg_hbm, tab_v)

        pltpu.sync_copy(flat_hbm.at[pl.ds(base, _SCHUNK)], idx_v)

        def body(i, carry):
            iv = idx_v[pl.ds(i * 16, 16)]
            val_v[pl.ds(i * 16, 16)] = plsc.load_gather(tab_v, [iv])
            return carry

        lax.fori_loop(0, _SCHUNK // 16, body, 0)
        pltpu.sync_copy(val_v, out_hbm.at[c, pl.ds(base, _SCHUNK)])

    return sc_gather


def _reduce_body(ge_ref, gg_ref, nif_ref, nib_ref, lab_ref, ii_ref, bias_ref,
                 acc_ref):
    i = pl.program_id(0)
    total = jnp.sum(nif_ref[...])
    j = lax.broadcasted_iota(jnp.int32, (1, _HN), 1)
    maskf = ((j + 1).astype(jnp.float32) <= total).astype(jnp.float32)
    denom = jnp.sum(ge_ref[...] * maskf, axis=1, keepdims=True)
    numer = jnp.sum(gg_ref[...] * maskf, axis=1, keepdims=True)
    nb = nib_ref[...]
    labs = lab_ref[...]
    coeff = 1.0 / jnp.sqrt(nb)
    dot = numer / jnp.sqrt(denom)
    u = lax.broadcasted_iota(jnp.int32, (1, 256), 1)
    onehot = (ii_ref[...] == u).astype(jnp.float32)
    biasv = jnp.sum(onehot * bias_ref[...], axis=1, keepdims=True)
    arg = coeff * dot + biasv
    out = 1.0 / (1.0 + jnp.exp(-arg))
    outc = jnp.clip(out, 1e-12, 1.0)
    t = labs * jnp.log(outc) + (1.0 - labs) * jnp.log(1.0 - outc)
    part = jnp.sum(t)

    @pl.when(i == 0)
    def _():
        acc_ref[0, 0] = 0.0

    acc_ref[0, 0] += part


_reduce_call = pl.pallas_call(
    _reduce_body,
    grid=(_ASTEPS,),
    in_specs=[
        pl.BlockSpec((_BBLK, _HN), lambda i: (i, 0)),
        pl.BlockSpec((_BBLK, _HN), lambda i: (i, 0)),
        pl.BlockSpec((1, _B), lambda i: (0, 0)),
        pl.BlockSpec((_BBLK, 1), lambda i: (i, 0)),
        pl.BlockSpec((_BBLK, 1), lambda i: (i, 0)),
        pl.BlockSpec((_BBLK, 1), lambda i: (i, 0)),
        pl.BlockSpec((1, 256), lambda i: (0, 0)),
    ],
    out_specs=pl.BlockSpec(memory_space=pltpu.SMEM, block_shape=(1, 1),
                           index_map=lambda i: (0, 0)),
    out_shape=jax.ShapeDtypeStruct((1, 1), jnp.float32),
)


_NBLK = 4000
_NSTEPS = _NI // _NBLK


def _norm_body(q_ref, q2_ref, ql_ref, sq_ref, sq2_ref):
    i = pl.program_id(0)

    @pl.when(i == 0)
    def _():
        sq_ref[0, 0] = 0.0
        ql = ql_ref[...]
        sq2_ref[0, 0] = jnp.sum(ql * ql)

    q = q_ref[...]
    q2 = q2_ref[...]
    sq_ref[0, 0] += jnp.sum(q * q)
    sq2_ref[0, 0] += jnp.sum(q2 * q2)


_norm_call = pl.pallas_call(
    _norm_body,
    grid=(_NSTEPS,),
    in_specs=[
        pl.BlockSpec((_NBLK, _D), lambda i: (i, 0)),
        pl.BlockSpec((_NBLK, _D), lambda i: (i, 0)),
        pl.BlockSpec((1, _D), lambda i: (0, 0)),
    ],
    out_specs=[
        pl.BlockSpec(memory_space=pltpu.SMEM, block_shape=(1, 1),
                     index_map=lambda i: (0, 0)),
        pl.BlockSpec(memory_space=pltpu.SMEM, block_shape=(1, 1),
                     index_map=lambda i: (0, 0)),
    ],
    out_shape=[
        jax.ShapeDtypeStruct((1, 1), jnp.float32),
        jax.ShapeDtypeStruct((1, 1), jnp.float32),
    ],
)


def kernel(user_input, num_idx, item_input, labels, embedding_Q,
           embedding_Q_, bias, W, b, h):
    ui = jnp.mod(user_input, 256).astype(jnp.int32)
    ii = jnp.mod(item_input, 256).astype(jnp.int32).reshape(-1, 1)
    e256 = embedding_Q[:256]
    bias256 = bias[:256].reshape(1, 256)
    hrow = h.reshape(1, _D)

    et, eg, flat, sw = _tables_call(e256, e256, W, b, hrow, ui, ii)
    gath = _make_sc_gather()(et.reshape(-1), eg.reshape(-1),
                             flat.reshape(-1))

    bsum = _reduce_call(gath[0].reshape(_B, _HN), gath[1].reshape(_B, _HN),
                        num_idx.reshape(1, _B),
                        num_idx.reshape(_B, 1), labels.reshape(_B, 1),
                        ii, bias256)
    ssq, ssq2 = _norm_call(embedding_Q, embedding_Q_[:_NI], embedding_Q_[_NI:])

    bce = -bsum[0, 0] / _B
    loss = (bce + _LAM * jnp.sqrt(ssq[0, 0]) + _GAM * jnp.sqrt(ssq2[0, 0])
            + _ETA * jnp.sqrt(sw[0, 0]))
    return loss


# speedup vs baseline: 9.0683x; 1.0956x over previous
import functools

import jax
import jax.numpy as jnp
from jax import lax
from jax.experimental import pallas as pl
from jax.experimental.pallas import tpu as pltpu
from jax.experimental.pallas import tpu_sc as plsc

_B = 4096
_HN = 50
_D = 64
_NI = 100000
_ALPHA = 0.5
_BETA = 0.5
_LAM = 1e-7
_GAM = 1e-7
_ETA = 1e-5

_UBLK = 32
_ASTEPS = 256 // _UBLK
_BBLK = _B // _ASTEPS

_NW = 32
_TOT = _B * _HN
_CHUNK = _TOT // _NW


def _tables_body(ef_ref, eb_ref, w_ref, b_ref, h_ref, ui_ref, ii_ref,
                 et_ref, eg_ref, flat_ref, sw_ref):
    i = pl.program_id(0)
    ef = ef_ref[...]
    eb = eb_ref[...]
    w = w_ref[...]
    p_all = lax.dot_general(ef, w, (((1,), (0,)), ((), ())),
                            precision=lax.Precision.HIGHEST,
                            preferred_element_type=jnp.float32)
    p_u = lax.dot_general(eb, w, (((1,), (0,)), ((), ())),
                          precision=lax.Precision.HIGHEST,
                          preferred_element_type=jnp.float32)
    pre = p_u[:, None, :] + p_all[None, :, :] + b_ref[...][None, :, :]
    sig = 1.0 / (1.0 + jnp.exp(-pre))
    t = jnp.sum(sig * h_ref[...][None, :, :], axis=2)
    et = jnp.exp(t)
    g = lax.dot_general(eb, ef, (((1,), (1,)), ((), ())),
                        precision=lax.Precision.HIGHEST,
                        preferred_element_type=jnp.float32)
    et_ref[...] = et
    eg_ref[...] = et * g
    flat_ref[...] = ui_ref[...] * 256 + ii_ref[...]

    @pl.when(i == 0)
    def _():
        sw_ref[0, 0] = jnp.sum(w * w)


_tables_call = pl.pallas_call(
    _tables_body,
    grid=(_ASTEPS,),
    in_specs=[
        pl.BlockSpec((256, _D), lambda i: (0, 0)),
        pl.BlockSpec((_UBLK, _D), lambda i: (i, 0)),
        pl.BlockSpec((_D, _D), lambda i: (0, 0)),
        pl.BlockSpec((1, _D), lambda i: (0, 0)),
        pl.BlockSpec((1, _D), lambda i: (0, 0)),
        pl.BlockSpec((_BBLK, _HN), lambda i: (i, 0)),
        pl.BlockSpec((_BBLK, 1), lambda i: (i, 0)),
    ],
    out_specs=[
        pl.BlockSpec((_UBLK, 256), lambda i: (i, 0)),
        pl.BlockSpec((_UBLK, 256), lambda i: (i, 0)),
        pl.BlockSpec((_BBLK, _HN), lambda i: (i, 0)),
        pl.BlockSpec(memory_space=pltpu.SMEM, block_shape=(1, 1),
                     index_map=lambda i: (0, 0)),
    ],
    out_shape=[
        jax.ShapeDtypeStruct((256, 256), jnp.float32),
        jax.ShapeDtypeStruct((256, 256), jnp.float32),
        jax.ShapeDtypeStruct((_B, _HN), jnp.int32),
        jax.ShapeDtypeStruct((1, 1), jnp.float32),
    ],
)


_SCHUNK = _TOT // 16


@functools.cache
def _make_sc_gather():
    mesh = plsc.VectorSubcoreMesh(core_axis_name="c", subcore_axis_name="s",
                                  num_cores=2, num_subcores=16)

    @functools.partial(
        pl.kernel,
        out_type=jax.ShapeDtypeStruct((2, _TOT), jnp.float32),
        mesh=mesh,
        scratch_types=[
            pltpu.VMEM((65536,), jnp.float32),
            pltpu.VMEM((_SCHUNK,), jnp.int32),
            pltpu.VMEM((_SCHUNK,), jnp.float32),
        ],
        compiler_params=pltpu.CompilerParams(use_tc_tiling_on_sc=False,
                                             needs_layout_passes=False),
    )
    def sc_gather(et_hbm, eg_hbm, flat_hbm, out_hbm, tab_v, idx_v, val_v):
        c = lax.axis_index("c")
        s = lax.axis_index("s")
        base = s * _SCHUNK

        @pl.when(c == 0)
        def _():
            pltpu.sync_copy(et_hbm, tab_v)

        @pl.when(c == 1)
        def _():
            pltpu.sync_copy(eg_hbm, tab_v)

        pltpu.sync_copy(flat_hbm.at[pl.ds(base, _SCHUNK)], idx_v)

        def body(i, carry):
            iv = idx_v[pl.ds(i * 16, 16)]
            val_v[pl.ds(i * 16, 16)] = plsc.load_gather(tab_v, [iv])
            return carry

        lax.fori_loop(0, _SCHUNK // 16, body, 0)
        pltpu.sync_copy(val_v, out_hbm.at[c, pl.ds(base, _SCHUNK)])

    return sc_gather


def _reduce_body(ge_ref, gg_ref, nif_ref, nib_ref, lab_ref, ii_ref, bias_ref,
                 acc_ref):
    i = pl.program_id(0)
    total = jnp.sum(nif_ref[...])
    j = lax.broadcasted_iota(jnp.int32, (1, _HN), 1)
    maskf = ((j + 1).astype(jnp.float32) <= total).astype(jnp.float32)
    denom = jnp.sum(ge_ref[...] * maskf, axis=1, keepdims=True)
    numer = jnp.sum(gg_ref[...] * maskf, axis=1, keepdims=True)
    nb = nib_ref[...]
    labs = lab_ref[...]
    coeff = 1.0 / jnp.sqrt(nb)
    dot = numer / jnp.sqrt(denom)
    u = lax.broadcasted_iota(jnp.int32, (1, 256), 1)
    onehot = (ii_ref[...] == u).astype(jnp.float32)
    biasv = jnp.sum(onehot * bias_ref[...], axis=1, keepdims=True)
    arg = coeff * dot + biasv
    out = 1.0 / (1.0 + jnp.exp(-arg))
    outc = jnp.clip(out, 1e-12, 1.0)
    t = labs * jnp.log(outc) + (1.0 - labs) * jnp.log(1.0 - outc)
    part = jnp.sum(t)

    @pl.when(i == 0)
    def _():
        acc_ref[0, 0] = 0.0

    acc_ref[0, 0] += part


_reduce_call = pl.pallas_call(
    _reduce_body,
    grid=(_ASTEPS,),
    in_specs=[
        pl.BlockSpec((_BBLK, _HN), lambda i: (i, 0)),
        pl.BlockSpec((_BBLK, _HN), lambda i: (i + _ASTEPS, 0)),
        pl.BlockSpec((1, _B), lambda i: (0, 0)),
        pl.BlockSpec((_BBLK, 1), lambda i: (i, 0)),
        pl.BlockSpec((_BBLK, 1), lambda i: (i, 0)),
        pl.BlockSpec((_BBLK, 1), lambda i: (i, 0)),
        pl.BlockSpec((1, 256), lambda i: (0, 0)),
    ],
    out_specs=pl.BlockSpec(memory_space=pltpu.SMEM, block_shape=(1, 1),
                           index_map=lambda i: (0, 0)),
    out_shape=jax.ShapeDtypeStruct((1, 1), jnp.float32),
)


_NBLK = 10000
_NSTEPS = _NI // _NBLK + 1


def _norm_body(q_ref, q2_ref, sq_ref, sq2_ref):
    i = pl.program_id(0)

    @pl.when(i == 0)
    def _():
        sq_ref[0, 0] = 0.0
        sq2_ref[0, 0] = 0.0

    @pl.when(i < _NSTEPS - 1)
    def _():
        q = q_ref[...]
        q2 = q2_ref[...]
        sq_ref[0, 0] += jnp.sum(q * q)
        sq2_ref[0, 0] += jnp.sum(q2 * q2)

    @pl.when(i == _NSTEPS - 1)
    def _():
        q2f = q2_ref[0:1, :]
        sq2_ref[0, 0] += jnp.sum(q2f * q2f)


_norm_call = pl.pallas_call(
    _norm_body,
    grid=(_NSTEPS,),
    in_specs=[
        pl.BlockSpec((_NBLK, _D), lambda i: (jnp.minimum(i, _NSTEPS - 2), 0)),
        pl.BlockSpec((_NBLK, _D), lambda i: (i, 0)),
    ],
    out_specs=[
        pl.BlockSpec(memory_space=pltpu.SMEM, block_shape=(1, 1),
                     index_map=lambda i: (0, 0)),
        pl.BlockSpec(memory_space=pltpu.SMEM, block_shape=(1, 1),
                     index_map=lambda i: (0, 0)),
    ],
    out_shape=[
        jax.ShapeDtypeStruct((1, 1), jnp.float32),
        jax.ShapeDtypeStruct((1, 1), jnp.float32),
    ],
)


def kernel(user_input, num_idx, item_input, labels, embedding_Q,
           embedding_Q_, bias, W, b, h):
    ui = jnp.mod(user_input, 256).astype(jnp.int32)
    ii = jnp.mod(item_input, 256).astype(jnp.int32).reshape(-1, 1)
    e256 = embedding_Q[:256]
    bias256 = bias[:256].reshape(1, 256)
    hrow = h.reshape(1, _D)

    et, eg, flat, sw = _tables_call(e256, e256, W, b, hrow, ui, ii)
    gath = _make_sc_gather()(et.reshape(-1), eg.reshape(-1),
                             flat.reshape(-1))
    g2 = gath.reshape(2 * _B, _HN)

    bsum = _reduce_call(g2, g2, num_idx.reshape(1, _B),
                        num_idx.reshape(_B, 1), labels.reshape(_B, 1),
                        ii, bias256)
    ssq, ssq2 = _norm_call(embedding_Q, embedding_Q_)

    bce = -bsum[0, 0] / _B
    loss = (bce + _LAM * jnp.sqrt(ssq[0, 0]) + _GAM * jnp.sqrt(ssq2[0, 0])
            + _ETA * jnp.sqrt(sw[0, 0]))
    return loss


# speedup vs baseline: 19.7083x; 2.1733x over previous
import functools

import jax
import jax.numpy as jnp
from jax import lax
from jax.experimental import pallas as pl
from jax.experimental.pallas import tpu as pltpu
from jax.experimental.pallas import tpu_sc as plsc

_B = 4096
_HN = 50
_D = 64
_NI = 100000
_ALPHA = 0.5
_BETA = 0.5
_LAM = 1e-7
_GAM = 1e-7
_ETA = 1e-5

_UBLK = 32
_ASTEPS = 256 // _UBLK
_BBLK = _B // _ASTEPS

_NW = 32
_TOT = _B * _HN
_CHUNK = _TOT // _NW


def _tables_body(ef_ref, w_ref, b_ref, h_ref, ui_ref, ii_ref,
                 et_ref, eg_ref, flat_ref, sw_ref, ps_ref, pt_ref, gs_ref):
    i = pl.program_id(0)

    @pl.when(i == 0)
    def _():
        ef = ef_ref[...]
        w = w_ref[...]
        ps_ref[...] = lax.dot_general(ef, w, (((0,), (0,)), ((), ())),
                                      precision=lax.Precision.HIGHEST,
                                      preferred_element_type=jnp.float32)
        pt_ref[...] = lax.dot_general(w, ef, (((0,), (0,)), ((), ())),
                                      precision=lax.Precision.HIGHEST,
                                      preferred_element_type=jnp.float32)
        gs_ref[...] = lax.dot_general(ef, ef, (((0,), (0,)), ((), ())),
                                      precision=lax.Precision.HIGHEST,
                                      preferred_element_type=jnp.float32)
        sw_ref[0, 0] = jnp.sum(w * w)

    p_u = ps_ref[pl.ds(i * _UBLK, _UBLK), :]
    pt = pt_ref[...]
    g = gs_ref[pl.ds(i * _UBLK, _UBLK), :]
    pre = p_u[:, :, None] + pt[None, :, :] + b_ref[...][None, :, :]
    sig = 1.0 / (1.0 + jnp.exp(-pre))
    t = jnp.sum(sig * h_ref[...][None, :, :], axis=1)
    et = jnp.exp(t)
    et_ref[...] = et
    eg_ref[...] = et * g
    flat_ref[...] = ui_ref[...] * 256 + ii_ref[...]


_tables_call = pl.pallas_call(
    _tables_body,
    grid=(_ASTEPS,),
    in_specs=[
        pl.BlockSpec((_D, 256), lambda i: (0, 0)),
        pl.BlockSpec((_D, _D), lambda i: (0, 0)),
        pl.BlockSpec((_D, 1), lambda i: (0, 0)),
        pl.BlockSpec((_D, 1), lambda i: (0, 0)),
        pl.BlockSpec((_BBLK, _HN), lambda i: (i, 0)),
        pl.BlockSpec((_BBLK, 1), lambda i: (i, 0)),
    ],
    out_specs=[
        pl.BlockSpec((_UBLK, 256), lambda i: (i, 0)),
        pl.BlockSpec((_UBLK, 256), lambda i: (i, 0)),
        pl.BlockSpec((_BBLK, _HN), lambda i: (i, 0)),
        pl.BlockSpec(memory_space=pltpu.SMEM, block_shape=(1, 1),
                     index_map=lambda i: (0, 0)),
    ],
    out_shape=[
        jax.ShapeDtypeStruct((256, 256), jnp.float32),
        jax.ShapeDtypeStruct((256, 256), jnp.float32),
        jax.ShapeDtypeStruct((_B, _HN), jnp.int32),
        jax.ShapeDtypeStruct((1, 1), jnp.float32),
    ],
    scratch_shapes=[
        pltpu.VMEM((256, _D), jnp.float32),
        pltpu.VMEM((_D, 256), jnp.float32),
        pltpu.VMEM((256, 256), jnp.float32),
    ],
)


_SCHUNK = _TOT // 16


@functools.cache
def _make_sc_gather():
    mesh = plsc.VectorSubcoreMesh(core_axis_name="c", subcore_axis_name="s",
                                  num_cores=2, num_subcores=16)

    @functools.partial(
        pl.kernel,
        out_type=jax.ShapeDtypeStruct((2, _TOT), jnp.float32),
        mesh=mesh,
        scratch_types=[
            pltpu.VMEM((65536,), jnp.float32),
            pltpu.VMEM((_SCHUNK,), jnp.int32),
            pltpu.VMEM((_SCHUNK,), jnp.float32),
        ],
        compiler_params=pltpu.CompilerParams(use_tc_tiling_on_sc=False,
                                             needs_layout_passes=False),
    )
    def sc_gather(et_hbm, eg_hbm, flat_hbm, out_hbm, tab_v, idx_v, val_v):
        c = lax.axis_index("c")
        s = lax.axis_index("s")
        base = s * _SCHUNK

        @pl.when(c == 0)
        def _():
            pltpu.sync_copy(et_hbm, tab_v)

        @pl.when(c == 1)
        def _():
            pltpu.sync_copy(eg_hbm, tab_v)

        pltpu.sync_copy(flat_hbm.at[pl.ds(base, _SCHUNK)], idx_v)

        def body(i, carry):
            for k in range(16):
                off = i * 256 + k * 16
                iv = idx_v[pl.ds(off, 16)]
                val_v[pl.ds(off, 16)] = plsc.load_gather(tab_v, [iv])
            return carry

        lax.fori_loop(0, _SCHUNK // 256, body, 0)
        pltpu.sync_copy(val_v, out_hbm.at[c, pl.ds(base, _SCHUNK)])

    return sc_gather


def _reduce_body(ge_ref, gg_ref, nif_ref, nib_ref, lab_ref, ii_ref, bias_ref,
                 acc_ref):
    i = pl.program_id(0)
    total = jnp.sum(nif_ref[...])
    j = lax.broadcasted_iota(jnp.int32, (1, _HN), 1)
    maskf = ((j + 1).astype(jnp.float32) <= total).astype(jnp.float32)
    denom = jnp.sum(ge_ref[...] * maskf, axis=1, keepdims=True)
    numer = jnp.sum(gg_ref[...] * maskf, axis=1, keepdims=True)
    nb = nib_ref[...]
    labs = lab_ref[...]
    coeff = 1.0 / jnp.sqrt(nb)
    dot = numer / jnp.sqrt(denom)
    u = lax.broadcasted_iota(jnp.int32, (1, 256), 1)
    onehot = (ii_ref[...] == u).astype(jnp.float32)
    biasv = jnp.sum(onehot * bias_ref[...], axis=1, keepdims=True)
    arg = coeff * dot + biasv
    out = 1.0 / (1.0 + jnp.exp(-arg))
    outc = jnp.clip(out, 1e-12, 1.0)
    t = labs * jnp.log(outc) + (1.0 - labs) * jnp.log(1.0 - outc)
    part = jnp.sum(t)

    @pl.when(i == 0)
    def _():
        acc_ref[0, 0] = 0.0

    acc_ref[0, 0] += part


_reduce_call = pl.pallas_call(
    _reduce_body,
    grid=(_ASTEPS,),
    in_specs=[
        pl.BlockSpec((_BBLK, _HN), lambda i: (i, 0)),
        pl.BlockSpec((_BBLK, _HN), lambda i: (i + _ASTEPS, 0)),
        pl.BlockSpec((1, _B), lambda i: (0, 0)),
        pl.BlockSpec((_BBLK, 1), lambda i: (i, 0)),
        pl.BlockSpec((_BBLK, 1), lambda i: (i, 0)),
        pl.BlockSpec((_BBLK, 1), lambda i: (i, 0)),
        pl.BlockSpec((1, 256), lambda i: (0, 0)),
    ],
    out_specs=pl.BlockSpec(memory_space=pltpu.SMEM, block_shape=(1, 1),
                           index_map=lambda i: (0, 0)),
    out_shape=jax.ShapeDtypeStruct((1, 1), jnp.float32),
)


_NBLK = 25600
_NSTEPS = 4


def _norm_body(q_ref, q2_ref, sq_ref, sq2_ref):
    i = pl.program_id(0)

    @pl.when(i == 0)
    def _():
        sq_ref[0, 0] = 0.0
        sq2_ref[0, 0] = 0.0

    @pl.when(i < _NSTEPS - 1)
    def _():
        q = q_ref[...]
        q2 = q2_ref[...]
        sq_ref[0, 0] += jnp.sum(q * q)
        sq2_ref[0, 0] += jnp.sum(q2 * q2)

    @pl.when(i == _NSTEPS - 1)
    def _():
        lane = lax.broadcasted_iota(jnp.int32, (_D, _NBLK), 1)
        q = q_ref[...]
        q2 = q2_ref[...]
        nq = _NI - (_NSTEPS - 1) * _NBLK
        sq_ref[0, 0] += jnp.sum(jnp.where(lane < nq, q * q, 0.0))
        sq2_ref[0, 0] += jnp.sum(jnp.where(lane < nq + 1, q2 * q2, 0.0))


_norm_call = pl.pallas_call(
    _norm_body,
    grid=(_NSTEPS,),
    in_specs=[
        pl.BlockSpec((_D, _NBLK), lambda i: (0, i)),
        pl.BlockSpec((_D, _NBLK), lambda i: (0, i)),
    ],
    out_specs=[
        pl.BlockSpec(memory_space=pltpu.SMEM, block_shape=(1, 1),
                     index_map=lambda i: (0, 0)),
        pl.BlockSpec(memory_space=pltpu.SMEM, block_shape=(1, 1),
                     index_map=lambda i: (0, 0)),
    ],
    out_shape=[
        jax.ShapeDtypeStruct((1, 1), jnp.float32),
        jax.ShapeDtypeStruct((1, 1), jnp.float32),
    ],
)


def kernel(user_input, num_idx, item_input, labels, embedding_Q,
           embedding_Q_, bias, W, b, h):
    ui = jnp.mod(user_input, 256).astype(jnp.int32)
    ii = jnp.mod(item_input, 256).astype(jnp.int32).reshape(-1, 1)
    e256t = embedding_Q.T[:, :256]
    bias256 = bias[:256].reshape(1, 256)

    et, eg, flat, sw = _tables_call(e256t, W, b.T, h, ui, ii)
    gath = _make_sc_gather()(et.reshape(-1), eg.reshape(-1),
                             flat.reshape(-1))
    g2 = gath.reshape(2 * _B, _HN)

    bsum = _reduce_call(g2, g2, num_idx.reshape(1, _B),
                        num_idx.reshape(_B, 1), labels.reshape(_B, 1),
                        ii, bias256)
    ssq, ssq2 = _norm_call(embedding_Q.T, embedding_Q_.T)

    bce = -bsum[0, 0] / _B
    loss = (bce + _LAM * jnp.sqrt(ssq[0, 0]) + _GAM * jnp.sqrt(ssq2[0, 0])
            + _ETA * jnp.sqrt(sw[0, 0]))
    return loss


# speedup vs baseline: 22.5664x; 1.1450x over previous
import functools

import jax
import jax.numpy as jnp
from jax import lax
from jax.experimental import pallas as pl
from jax.experimental.pallas import tpu as pltpu
from jax.experimental.pallas import tpu_sc as plsc

_B = 4096
_HN = 50
_D = 64
_NI = 100000
_ALPHA = 0.5
_BETA = 0.5
_LAM = 1e-7
_GAM = 1e-7
_ETA = 1e-5

_UBLK = 32
_ASTEPS = 256 // _UBLK
_BBLK = _B // _ASTEPS

_NW = 32
_TOT = _B * _HN
_CHUNK = _TOT // _NW


def _tables_body(ef_ref, w_ref, b_ref, h_ref, ui_ref, ii_ref,
                 et_ref, eg_ref, flat_ref, sw_ref, ps_ref, pt_ref, gs_ref):
    i = pl.program_id(0)

    @pl.when(i == 0)
    def _():
        ef = ef_ref[...]
        w = w_ref[...]
        ps_ref[...] = lax.dot_general(ef, w, (((0,), (0,)), ((), ())),
                                      precision=lax.Precision.HIGHEST,
                                      preferred_element_type=jnp.float32)
        pt_ref[...] = lax.dot_general(w, ef, (((0,), (0,)), ((), ())),
                                      precision=lax.Precision.HIGHEST,
                                      preferred_element_type=jnp.float32)
        gs_ref[...] = lax.dot_general(ef, ef, (((0,), (0,)), ((), ())),
                                      precision=lax.Precision.HIGHEST,
                                      preferred_element_type=jnp.float32)
        sw_ref[0, 0] = jnp.sum(w * w)

    p_u = ps_ref[pl.ds(i * _UBLK, _UBLK), :]
    pt = pt_ref[...]
    g = gs_ref[pl.ds(i * _UBLK, _UBLK), :]
    pre = p_u[:, :, None] + pt[None, :, :] + b_ref[...][None, :, :]
    sig = 1.0 / (1.0 + jnp.exp(-pre))
    t = jnp.sum(sig * h_ref[...][None, :, :], axis=1)
    et = jnp.exp(t)
    et_ref[...] = et
    eg_ref[...] = et * g
    flat_ref[...] = ui_ref[...] * 256 + ii_ref[...]


_tables_call = pl.pallas_call(
    _tables_body,
    grid=(_ASTEPS,),
    in_specs=[
        pl.BlockSpec((_D, 256), lambda i: (0, 0)),
        pl.BlockSpec((_D, _D), lambda i: (0, 0)),
        pl.BlockSpec((_D, 1), lambda i: (0, 0)),
        pl.BlockSpec((_D, 1), lambda i: (0, 0)),
        pl.BlockSpec((_HN, _BBLK), lambda i: (0, i)),
        pl.BlockSpec((1, _BBLK), lambda i: (0, i)),
    ],
    out_specs=[
        pl.BlockSpec((_UBLK, 256), lambda i: (i, 0)),
        pl.BlockSpec((_UBLK, 256), lambda i: (i, 0)),
        pl.BlockSpec((_HN, _BBLK), lambda i: (0, i)),
        pl.BlockSpec(memory_space=pltpu.SMEM, block_shape=(1, 1),
                     index_map=lambda i: (0, 0)),
    ],
    out_shape=[
        jax.ShapeDtypeStruct((256, 256), jnp.float32),
        jax.ShapeDtypeStruct((256, 256), jnp.float32),
        jax.ShapeDtypeStruct((_HN, _B), jnp.int32),
        jax.ShapeDtypeStruct((1, 1), jnp.float32),
    ],
    scratch_shapes=[
        pltpu.VMEM((256, _D), jnp.float32),
        pltpu.VMEM((_D, 256), jnp.float32),
        pltpu.VMEM((256, 256), jnp.float32),
    ],
)


_SCHUNK = _TOT // 16


@functools.cache
def _make_sc_gather():
    mesh = plsc.VectorSubcoreMesh(core_axis_name="c", subcore_axis_name="s",
                                  num_cores=2, num_subcores=16)

    @functools.partial(
        pl.kernel,
        out_type=jax.ShapeDtypeStruct((2, _TOT), jnp.float32),
        mesh=mesh,
        scratch_types=[
            pltpu.VMEM((65536,), jnp.float32),
            pltpu.VMEM((_SCHUNK,), jnp.int32),
            pltpu.VMEM((_SCHUNK,), jnp.float32),
        ],
        compiler_params=pltpu.CompilerParams(use_tc_tiling_on_sc=False,
                                             needs_layout_passes=False),
    )
    def sc_gather(et_hbm, eg_hbm, flat_hbm, out_hbm, tab_v, idx_v, val_v):
        c = lax.axis_index("c")
        s = lax.axis_index("s")
        base = s * _SCHUNK

        @pl.when(c == 0)
        def _():
            pltpu.sync_copy(et_hbm, tab_v)

        @pl.when(c == 1)
        def _():
            pltpu.sync_copy(eg_hbm, tab_v)

        pltpu.sync_copy(flat_hbm.at[pl.ds(base, _SCHUNK)], idx_v)

        def body(i, carry):
            for k in range(16):
                off = i * 256 + k * 16
                iv = idx_v[pl.ds(off, 16)]
                val_v[pl.ds(off, 16)] = plsc.load_gather(tab_v, [iv])
            return carry

        lax.fori_loop(0, _SCHUNK // 256, body, 0)
        pltpu.sync_copy(val_v, out_hbm.at[c, pl.ds(base, _SCHUNK)])

    return sc_gather


def _reduce_body(ge_ref, gg_ref, nif_ref, nb_ref, lab_ref, ii_ref, bias_ref,
                 acc_ref):
    i = pl.program_id(0)
    total = jnp.sum(nif_ref[...])
    jj = lax.broadcasted_iota(jnp.int32, (_HN, 1), 0)
    maskf = ((jj + 1).astype(jnp.float32) <= total).astype(jnp.float32)
    ge = ge_ref[...][0]
    gg = gg_ref[...][0]
    denom = jnp.sum(ge * maskf, axis=0, keepdims=True)
    numer = jnp.sum(gg * maskf, axis=0, keepdims=True)
    nb = nb_ref[...]
    labs = lab_ref[...]
    coeff = 1.0 / jnp.sqrt(nb)
    dot = numer / jnp.sqrt(denom)
    uu = lax.broadcasted_iota(jnp.int32, (256, 1), 0)
    onehot = (ii_ref[...] == uu).astype(jnp.float32)
    biasv = jnp.sum(onehot * bias_ref[...], axis=0, keepdims=True)
    arg = coeff * dot + biasv
    out = 1.0 / (1.0 + jnp.exp(-arg))
    outc = jnp.clip(out, 1e-12, 1.0)
    t = labs * jnp.log(outc) + (1.0 - labs) * jnp.log(1.0 - outc)
    part = jnp.sum(t)

    @pl.when(i == 0)
    def _():
        acc_ref[0, 0] = 0.0

    acc_ref[0, 0] += part


_reduce_call = pl.pallas_call(
    _reduce_body,
    grid=(_ASTEPS,),
    in_specs=[
        pl.BlockSpec((1, _HN, _BBLK), lambda i: (0, 0, i)),
        pl.BlockSpec((1, _HN, _BBLK), lambda i: (1, 0, i)),
        pl.BlockSpec((1, _B), lambda i: (0, 0)),
        pl.BlockSpec((1, _BBLK), lambda i: (0, i)),
        pl.BlockSpec((1, _BBLK), lambda i: (0, i)),
        pl.BlockSpec((1, _BBLK), lambda i: (0, i)),
        pl.BlockSpec((256, 1), lambda i: (0, 0)),
    ],
    out_specs=pl.BlockSpec(memory_space=pltpu.SMEM, block_shape=(1, 1),
                           index_map=lambda i: (0, 0)),
    out_shape=jax.ShapeDtypeStruct((1, 1), jnp.float32),
)


_NBLK = 25600
_NSTEPS = 4


def _norm_body(q_ref, q2_ref, sq_ref, sq2_ref):
    i = pl.program_id(0)

    @pl.when(i == 0)
    def _():
        sq_ref[0, 0] = 0.0
        sq2_ref[0, 0] = 0.0

    @pl.when(i < _NSTEPS - 1)
    def _():
        q = q_ref[...]
        q2 = q2_ref[...]
        sq_ref[0, 0] += jnp.sum(q * q)
        sq2_ref[0, 0] += jnp.sum(q2 * q2)

    @pl.when(i == _NSTEPS - 1)
    def _():
        lane = lax.broadcasted_iota(jnp.int32, (_D, _NBLK), 1)
        q = q_ref[...]
        q2 = q2_ref[...]
        nq = _NI - (_NSTEPS - 1) * _NBLK
        sq_ref[0, 0] += jnp.sum(jnp.where(lane < nq, q * q, 0.0))
        sq2_ref[0, 0] += jnp.sum(jnp.where(lane < nq + 1, q2 * q2, 0.0))


_norm_call = pl.pallas_call(
    _norm_body,
    grid=(_NSTEPS,),
    in_specs=[
        pl.BlockSpec((_D, _NBLK), lambda i: (0, i)),
        pl.BlockSpec((_D, _NBLK), lambda i: (0, i)),
    ],
    out_specs=[
        pl.BlockSpec(memory_space=pltpu.SMEM, block_shape=(1, 1),
                     index_map=lambda i: (0, 0)),
        pl.BlockSpec(memory_space=pltpu.SMEM, block_shape=(1, 1),
                     index_map=lambda i: (0, 0)),
    ],
    out_shape=[
        jax.ShapeDtypeStruct((1, 1), jnp.float32),
        jax.ShapeDtypeStruct((1, 1), jnp.float32),
    ],
)


def kernel(user_input, num_idx, item_input, labels, embedding_Q,
           embedding_Q_, bias, W, b, h):
    uit = jnp.mod(user_input, 256).astype(jnp.int32).T
    iit = jnp.mod(item_input, 256).astype(jnp.int32).reshape(1, _B)
    e256t = embedding_Q.T[:, :256]
    biascol = bias[:256].reshape(256, 1)

    et, eg, flat, sw = _tables_call(e256t, W, b.T, h, uit, iit)
    gath = _make_sc_gather()(et.reshape(-1), eg.reshape(-1),
                             flat.reshape(-1))
    g3 = gath.reshape(2, _HN, _B)

    bsum = _reduce_call(g3, g3, num_idx.reshape(1, _B),
                        num_idx.reshape(1, _B), labels.reshape(1, _B),
                        iit, biascol)
    ssq, ssq2 = _norm_call(embedding_Q.T, embedding_Q_.T)

    bce = -bsum[0, 0] / _B
    loss = (bce + _LAM * jnp.sqrt(ssq[0, 0]) + _GAM * jnp.sqrt(ssq2[0, 0])
            + _ETA * jnp.sqrt(sw[0, 0]))
    return loss


# speedup vs baseline: 22.9442x; 1.0167x over previous
import functools

import jax
import jax.numpy as jnp
from jax import lax
from jax.experimental import pallas as pl
from jax.experimental.pallas import tpu as pltpu
from jax.experimental.pallas import tpu_sc as plsc

_B = 4096
_HN = 50
_D = 64
_NI = 100000
_ALPHA = 0.5
_BETA = 0.5
_LAM = 1e-7
_GAM = 1e-7
_ETA = 1e-5

_UBLK = 32
_ASTEPS = 256 // _UBLK
_BBLK = _B // _ASTEPS

_NW = 32
_TOT = _B * _HN
_CHUNK = _TOT // _NW


def _tables_body(ef_ref, w_ref, b_ref, h_ref, ui_ref, ii_ref,
                 et_ref, eg_ref, flat_ref, sw_ref, ps_ref, pt_ref, gs_ref):
    i = pl.program_id(0)

    @pl.when(i == 0)
    def _():
        ef = ef_ref[...]
        w = w_ref[...]
        ps_ref[...] = lax.dot_general(ef, w, (((0,), (0,)), ((), ())),
                                      precision=lax.Precision.HIGHEST,
                                      preferred_element_type=jnp.float32)
        pt_ref[...] = lax.dot_general(w, ef, (((0,), (0,)), ((), ())),
                                      precision=lax.Precision.HIGHEST,
                                      preferred_element_type=jnp.float32)
        gs_ref[...] = lax.dot_general(ef, ef, (((0,), (0,)), ((), ())),
                                      precision=lax.Precision.HIGHEST,
                                      preferred_element_type=jnp.float32)
        sw_ref[0, 0] = jnp.sum(w * w)

    p_u = ps_ref[pl.ds(i * _UBLK, _UBLK), :]
    pt = pt_ref[...]
    g = gs_ref[pl.ds(i * _UBLK, _UBLK), :]
    pre = p_u[:, :, None] + pt[None, :, :] + b_ref[...][None, :, :]
    sig = 1.0 / (1.0 + jnp.exp(-pre))
    t = jnp.sum(sig * h_ref[...][None, :, :], axis=1)
    et = jnp.exp(t)
    et_ref[...] = et
    eg_ref[...] = et * g
    flat_ref[...] = ui_ref[...] * 256 + ii_ref[...]


_tables_call = pl.pallas_call(
    _tables_body,
    grid=(_ASTEPS,),
    in_specs=[
        pl.BlockSpec((_D, 256), lambda i: (0, 0)),
        pl.BlockSpec((_D, _D), lambda i: (0, 0)),
        pl.BlockSpec((_D, 1), lambda i: (0, 0)),
        pl.BlockSpec((_D, 1), lambda i: (0, 0)),
        pl.BlockSpec((_HN, _BBLK), lambda i: (0, i)),
        pl.BlockSpec((1, _BBLK), lambda i: (0, i)),
    ],
    out_specs=[
        pl.BlockSpec((_UBLK, 256), lambda i: (i, 0)),
        pl.BlockSpec((_UBLK, 256), lambda i: (i, 0)),
        pl.BlockSpec((_HN, _BBLK), lambda i: (0, i)),
        pl.BlockSpec(memory_space=pltpu.SMEM, block_shape=(1, 1),
                     index_map=lambda i: (0, 0)),
    ],
    out_shape=[
        jax.ShapeDtypeStruct((256, 256), jnp.float32),
        jax.ShapeDtypeStruct((256, 256), jnp.float32),
        jax.ShapeDtypeStruct((_HN, _B), jnp.int32),
        jax.ShapeDtypeStruct((1, 1), jnp.float32),
    ],
    scratch_shapes=[
        pltpu.VMEM((256, _D), jnp.float32),
        pltpu.VMEM((_D, 256), jnp.float32),
        pltpu.VMEM((256, 256), jnp.float32),
    ],
)


_SCHUNK = _TOT // 16


@functools.cache
def _make_sc_gather():
    mesh = plsc.VectorSubcoreMesh(core_axis_name="c", subcore_axis_name="s",
                                  num_cores=2, num_subcores=16)

    @functools.partial(
        pl.kernel,
        out_type=jax.ShapeDtypeStruct((2, _TOT), jnp.float32),
        mesh=mesh,
        scratch_types=[
            pltpu.VMEM((65536,), jnp.float32),
            pltpu.VMEM((_SCHUNK,), jnp.int32),
            pltpu.VMEM((_SCHUNK,), jnp.float32),
        ],
        compiler_params=pltpu.CompilerParams(use_tc_tiling_on_sc=False,
                                             needs_layout_passes=False),
    )
    def sc_gather(et_hbm, eg_hbm, flat_hbm, out_hbm, tab_v, idx_v, val_v):
        c = lax.axis_index("c")
        s = lax.axis_index("s")
        base = s * _SCHUNK

        @pl.when(c == 0)
        def _():
            pltpu.sync_copy(et_hbm, tab_v)

        @pl.when(c == 1)
        def _():
            pltpu.sync_copy(eg_hbm, tab_v)

        pltpu.sync_copy(flat_hbm.at[pl.ds(base, _SCHUNK)], idx_v)

        def body(i, carry):
            for k in range(16):
                off = i * 256 + k * 16
                iv = idx_v[pl.ds(off, 16)]
                val_v[pl.ds(off, 16)] = plsc.load_gather(tab_v, [iv])
            return carry

        lax.fori_loop(0, _SCHUNK // 256, body, 0)
        pltpu.sync_copy(val_v, out_hbm.at[c, pl.ds(base, _SCHUNK)])

    return sc_gather


def _reduce_body(ge_ref, gg_ref, nif_ref, nb_ref, lab_ref, ii_ref, bias_ref,
                 sw_ref, sq_ref, sq2_ref, loss_ref, acc_ref):
    i = pl.program_id(0)
    total = jnp.sum(nif_ref[...])
    jj = lax.broadcasted_iota(jnp.int32, (_HN, 1), 0)
    maskf = ((jj + 1).astype(jnp.float32) <= total).astype(jnp.float32)
    ge = ge_ref[...][0]
    gg = gg_ref[...][0]
    denom = jnp.sum(ge * maskf, axis=0, keepdims=True)
    numer = jnp.sum(gg * maskf, axis=0, keepdims=True)
    nb = nb_ref[...]
    labs = lab_ref[...]
    coeff = 1.0 / jnp.sqrt(nb)
    dot = numer / jnp.sqrt(denom)
    uu = lax.broadcasted_iota(jnp.int32, (256, 1), 0)
    onehot = (ii_ref[...] == uu).astype(jnp.float32)
    biasv = jnp.sum(onehot * bias_ref[...], axis=0, keepdims=True)
    arg = coeff * dot + biasv
    out = 1.0 / (1.0 + jnp.exp(-arg))
    outc = jnp.clip(out, 1e-12, 1.0)
    t = labs * jnp.log(outc) + (1.0 - labs) * jnp.log(1.0 - outc)
    part = jnp.sum(t)

    @pl.when(i == 0)
    def _():
        acc_ref[0, 0] = 0.0

    acc_ref[0, 0] += part

    @pl.when(i == _ASTEPS - 1)
    def _():
        loss_ref[0, 0] = (-acc_ref[0, 0] / _B
                          + _LAM * jnp.sqrt(sq_ref[0, 0])
                          + _GAM * jnp.sqrt(sq2_ref[0, 0])
                          + _ETA * jnp.sqrt(sw_ref[0, 0]))


_reduce_call = pl.pallas_call(
    _reduce_body,
    grid=(_ASTEPS,),
    in_specs=[
        pl.BlockSpec((1, _HN, _BBLK), lambda i: (0, 0, i)),
        pl.BlockSpec((1, _HN, _BBLK), lambda i: (1, 0, i)),
        pl.BlockSpec((1, _B), lambda i: (0, 0)),
        pl.BlockSpec((1, _BBLK), lambda i: (0, i)),
        pl.BlockSpec((1, _BBLK), lambda i: (0, i)),
        pl.BlockSpec((1, _BBLK), lambda i: (0, i)),
        pl.BlockSpec((256, 1), lambda i: (0, 0)),
        pl.BlockSpec(memory_space=pltpu.SMEM, block_shape=(1, 1),
                     index_map=lambda i: (0, 0)),
        pl.BlockSpec(memory_space=pltpu.SMEM, block_shape=(1, 1),
                     index_map=lambda i: (0, 0)),
        pl.BlockSpec(memory_space=pltpu.SMEM, block_shape=(1, 1),
                     index_map=lambda i: (0, 0)),
    ],
    out_specs=pl.BlockSpec(memory_space=pltpu.SMEM, block_shape=(1, 1),
                           index_map=lambda i: (0, 0)),
    out_shape=jax.ShapeDtypeStruct((1, 1), jnp.float32),
    scratch_shapes=[pltpu.SMEM((1, 1), jnp.float32)],
)


_NBLK = 12800
_NSTEPS = 8


def _norm_body(q_ref, q2_ref, sq_ref, sq2_ref):
    i = pl.program_id(0)

    @pl.when(i == 0)
    def _():
        sq_ref[0, 0] = 0.0
        sq2_ref[0, 0] = 0.0

    @pl.when(i < _NSTEPS - 1)
    def _():
        q = q_ref[...]
        q2 = q2_ref[...]
        sq_ref[0, 0] += jnp.sum(q * q)
        sq2_ref[0, 0] += jnp.sum(q2 * q2)

    @pl.when(i == _NSTEPS - 1)
    def _():
        lane = lax.broadcasted_iota(jnp.int32, (_D, _NBLK), 1)
        q = q_ref[...]
        q2 = q2_ref[...]
        nq = _NI - (_NSTEPS - 1) * _NBLK
        sq_ref[0, 0] += jnp.sum(jnp.where(lane < nq, q * q, 0.0))
        sq2_ref[0, 0] += jnp.sum(jnp.where(lane < nq + 1, q2 * q2, 0.0))


_norm_call = pl.pallas_call(
    _norm_body,
    grid=(_NSTEPS,),
    in_specs=[
        pl.BlockSpec((_D, _NBLK), lambda i: (0, i)),
        pl.BlockSpec((_D, _NBLK), lambda i: (0, i)),
    ],
    out_specs=[
        pl.BlockSpec(memory_space=pltpu.SMEM, block_shape=(1, 1),
                     index_map=lambda i: (0, 0)),
        pl.BlockSpec(memory_space=pltpu.SMEM, block_shape=(1, 1),
                     index_map=lambda i: (0, 0)),
    ],
    out_shape=[
        jax.ShapeDtypeStruct((1, 1), jnp.float32),
        jax.ShapeDtypeStruct((1, 1), jnp.float32),
    ],
)


def kernel(user_input, num_idx, item_input, labels, embedding_Q,
           embedding_Q_, bias, W, b, h):
    uit = jnp.mod(user_input, 256).astype(jnp.int32).T
    iit = jnp.mod(item_input, 256).astype(jnp.int32).reshape(1, _B)
    e256t = embedding_Q.T[:, :256]
    biascol = bias[:256].reshape(256, 1)

    et, eg, flat, sw = _tables_call(e256t, W, b.T, h, uit, iit)
    gath = _make_sc_gather()(et.reshape(-1), eg.reshape(-1),
                             flat.reshape(-1))
    g3 = gath.reshape(2, _HN, _B)
    ssq, ssq2 = _norm_call(embedding_Q.T, embedding_Q_.T)

    loss = _reduce_call(g3, g3, num_idx.reshape(1, _B),
                        num_idx.reshape(1, _B), labels.reshape(1, _B),
                        iit, biascol, sw, ssq, ssq2)
    return loss[0, 0]
